# baseline (speedup 1.0000x reference)
import functools

import jax
import jax.numpy as jnp
from jax import lax
from jax.experimental import pallas as pl
from jax.experimental.pallas import tpu as pltpu
from jax.experimental.pallas import tpu_sc as plsc

N = 10000
D = 128
E = 320000
NP = 10240
NC, NS = 2, 16
NW = NC * NS
CHUNK = 128
RPT = 80
NPASS = 2
HP = RPT // NPASS
EP = NW * RPT * CHUNK
ROWS_PER_SUB = NP // NS

_MESH = plsc.VectorSubcoreMesh(core_axis_name="c", subcore_axis_name="s")


@functools.partial(
    pl.kernel,
    out_type=jax.ShapeDtypeStruct((NC * NP, D), jnp.float32),
    mesh=_MESH,
    scratch_types=[
        pltpu.VMEM((HP + 1, CHUNK), jnp.int32),
        pltpu.VMEM((HP, CHUNK), jnp.int32),
        pltpu.VMEM((CHUNK, D), jnp.float32),
        pltpu.VMEM((CHUNK, D), jnp.float32),
        pltpu.VMEM_SHARED((NP, D), jnp.float32),
        pltpu.SemaphoreType.DMA,
        pltpu.SemaphoreType.DMA,
    ],
)
def _sc_agg(hs_hbm, src_hbm, dst_hbm, zeros_hbm, out_hbm,
            src_v, dst_v, rows0_v, rows1_v, agg_sh, sem0, sem1):
    cid = lax.axis_index("c")
    sid = lax.axis_index("s")
    wid = cid * NS + sid
    pltpu.sync_copy(zeros_hbm.at[pl.ds(sid * ROWS_PER_SUB, ROWS_PER_SUB)],
                    agg_sh.at[pl.ds(sid * ROWS_PER_SUB, ROWS_PER_SUB)])
    plsc.subcore_barrier()

    for p in range(NPASS):
        pltpu.sync_copy(src_hbm.at[wid * NPASS + p], src_v)
        pltpu.sync_copy(dst_hbm.at[wid * NPASS + p], dst_v)

        pltpu.async_copy(hs_hbm.at[src_v.at[0]], rows0_v, sem0)

        def body(g, carry):
            j = 2 * g
            pltpu.async_copy(hs_hbm.at[src_v.at[j + 1]], rows1_v, sem1)
            pltpu.make_async_copy(hs_hbm.at[src_v.at[j]], rows0_v, sem0).wait()
            pltpu.sync_copy(rows0_v, agg_sh.at[dst_v.at[j]], add=True)
            pltpu.async_copy(hs_hbm.at[src_v.at[j + 2]], rows0_v, sem0)
            pltpu.make_async_copy(
                hs_hbm.at[src_v.at[j + 1]], rows1_v, sem1).wait()
            pltpu.sync_copy(rows1_v, agg_sh.at[dst_v.at[j + 1]], add=True)
            return carry

        lax.fori_loop(0, HP // 2, body, 0)
        pltpu.make_async_copy(hs_hbm.at[src_v.at[HP]], rows0_v, sem0).wait()
    plsc.subcore_barrier()
    off = cid * NP + sid * ROWS_PER_SUB
    pltpu.sync_copy(agg_sh.at[pl.ds(sid * ROWS_PER_SUB, ROWS_PER_SUB)],
                    out_hbm.at[pl.ds(off, ROWS_PER_SUB)])


@functools.partial(
    pl.kernel,
    out_type=jax.ShapeDtypeStruct((NC * NP, 8), jnp.float32),
    mesh=_MESH,
    scratch_types=[
        pltpu.VMEM((HP, CHUNK), jnp.int32),
        pltpu.VMEM((CHUNK, 8), jnp.float32),
        pltpu.VMEM_SHARED((NP, 8), jnp.float32),
    ],
)
def _sc_deg(dst_hbm, ones_hbm, zeros8_hbm, out_hbm, dst_v, ones_v, deg_sh):
    cid = lax.axis_index("c")
    sid = lax.axis_index("s")
    wid = cid * NS + sid
    pltpu.sync_copy(zeros8_hbm.at[pl.ds(sid * ROWS_PER_SUB, ROWS_PER_SUB)],
                    deg_sh.at[pl.ds(sid * ROWS_PER_SUB, ROWS_PER_SUB)])
    pltpu.sync_copy(ones_hbm, ones_v)
    plsc.subcore_barrier()

    def body(j, carry):
        pltpu.sync_copy(ones_v, deg_sh.at[dst_v.at[j]], add=True)
        return carry

    for p in range(NPASS):
        pltpu.sync_copy(dst_hbm.at[wid * NPASS + p], dst_v)
        lax.fori_loop(0, HP, body, 0)
    plsc.subcore_barrier()
    off = cid * NP + sid * ROWS_PER_SUB
    pltpu.sync_copy(deg_sh.at[pl.ds(sid * ROWS_PER_SUB, ROWS_PER_SUB)],
                    out_hbm.at[pl.ds(off, ROWS_PER_SUB)])


_R = 512


def _dinv_of(deg2_ref):
    deg = deg2_ref[0, :, 0:1] + deg2_ref[1, :, 0:1] + 1.0
    return lax.rsqrt(deg)


def _pre_body(x_ref, w_ref, deg2_ref, o_ref):
    dinv = _dinv_of(deg2_ref)
    h = jnp.dot(x_ref[...], w_ref[...], preferred_element_type=jnp.float32)
    o_ref[...] = h * dinv


def _mid_body(agg_ref, hs_ref, deg2_ref, w_ref, b_ref, o_ref):
    dinv = _dinv_of(deg2_ref)
    t = (agg_ref[0] + agg_ref[1] + hs_ref[...]) * dinv + b_ref[...]
    t = jnp.maximum(t, 0.0)
    nrm = jnp.sqrt(jnp.sum(t * t, axis=1, keepdims=True))
    t = t / jnp.clip(nrm, 1e-12, None)
    o_ref[...] = jnp.dot(t, w_ref[...], preferred_element_type=jnp.float32) * dinv


def _fin_body(agg_ref, hs_ref, deg2_ref, b_ref, o_ref):
    dinv = _dinv_of(deg2_ref)
    o_ref[...] = (agg_ref[0] + agg_ref[1] + hs_ref[...]) * dinv + b_ref[...]


_ROW_SPEC = pl.BlockSpec((_R, D), lambda i: (i, 0))
_AGG_SPEC = pl.BlockSpec((NC, _R, D), lambda i: (0, i, 0))
_DEG_SPEC = pl.BlockSpec((NC, _R, 8), lambda i: (0, i, 0))
_W_SPEC = pl.BlockSpec((D, D), lambda i: (0, 0))
_B_SPEC = pl.BlockSpec((1, D), lambda i: (0, 0))
_GRID = (NP // _R,)
_OUT = jax.ShapeDtypeStruct((NP, D), jnp.float32)

_tc_pre = pl.pallas_call(
    _pre_body, grid=_GRID, out_shape=_OUT,
    in_specs=[_ROW_SPEC, _W_SPEC, _DEG_SPEC], out_specs=_ROW_SPEC)

_tc_mid = pl.pallas_call(
    _mid_body, grid=_GRID, out_shape=_OUT,
    in_specs=[_AGG_SPEC, _ROW_SPEC, _DEG_SPEC, _W_SPEC, _B_SPEC],
    out_specs=_ROW_SPEC)

_tc_fin = pl.pallas_call(
    _fin_body, grid=_GRID, out_shape=_OUT,
    in_specs=[_AGG_SPEC, _ROW_SPEC, _DEG_SPEC, _B_SPEC], out_specs=_ROW_SPEC)


def kernel(x, edge_index, W1, b1, W2, b2, W3, b3):
    src = edge_index[0].astype(jnp.int32)
    dst = edge_index[1].astype(jnp.int32)
    pad = jnp.full((EP - E,), N, jnp.int32)
    src_t = jnp.concatenate([src, pad]).reshape(NW, NPASS, HP, CHUNK)
    src_t = jnp.concatenate(
        [src_t, jnp.full((NW, NPASS, 1, CHUNK), N, jnp.int32)],
        axis=2).reshape(NW * NPASS, HP + 1, CHUNK)
    dst_t = jnp.concatenate([dst, pad]).reshape(NW * NPASS, HP, CHUNK)
    x_pad = jnp.pad(x, ((0, NP - N), (0, 0)))

    zeros = jnp.zeros((NP, D), jnp.float32)
    zeros8 = jnp.zeros((NP, 8), jnp.float32)
    ones8 = jnp.ones((CHUNK, 8), jnp.float32)

    deg2 = _sc_deg(dst_t, ones8, zeros8).reshape(NC, NP, 8)

    hs = _tc_pre(x_pad, W1, deg2)
    agg = _sc_agg(hs, src_t, dst_t, zeros).reshape(NC, NP, D)
    hs = _tc_mid(agg, hs, deg2, W2, b1.reshape(1, D))
    agg = _sc_agg(hs, src_t, dst_t, zeros).reshape(NC, NP, D)
    hs = _tc_mid(agg, hs, deg2, W3, b2.reshape(1, D))
    agg = _sc_agg(hs, src_t, dst_t, zeros).reshape(NC, NP, D)
    out = _tc_fin(agg, hs, deg2, b3.reshape(1, D))
    return out[:N]


# speedup vs baseline: 2.8426x; 2.8426x over previous
import functools

import jax
import jax.numpy as jnp
from jax import lax
from jax.experimental import pallas as pl
from jax.experimental.pallas import tpu as pltpu
from jax.experimental.pallas import tpu_sc as plsc

N = 10000
D = 128
E = 320000
NP = 10240
NC, NS = 2, 16
NW = NC * NS
CHUNK = 128
RC0 = 100
RC1 = 57
STRIDE = RC0
EP = NW * STRIDE * CHUNK
ROWS_PER_SUB = NP // NS

_MESH = plsc.VectorSubcoreMesh(core_axis_name="c", subcore_axis_name="s")


@functools.partial(
    pl.kernel,
    out_type=jax.ShapeDtypeStruct((NC * NP, D), jnp.float32),
    mesh=_MESH,
    scratch_types=[
        pltpu.VMEM((STRIDE, CHUNK), jnp.int32),
        pltpu.VMEM((STRIDE, CHUNK), jnp.int32),
        pltpu.VMEM((CHUNK, D), jnp.float32),
        pltpu.VMEM_SHARED((NP, D), jnp.float32),
        pltpu.SemaphoreType.DMA,
    ],
)
def _sc_agg(hs_hbm, src_hbm, dst_hbm, zeros_hbm, out_hbm,
            src_v, dst_v, rows_v, agg_sh, sem):
    cid = lax.axis_index("c")
    sid = lax.axis_index("s")
    wid = cid * NS + sid
    pltpu.sync_copy(zeros_hbm.at[pl.ds(sid * ROWS_PER_SUB, ROWS_PER_SUB)],
                    agg_sh.at[pl.ds(sid * ROWS_PER_SUB, ROWS_PER_SUB)])
    pltpu.sync_copy(src_hbm.at[wid], src_v)
    pltpu.sync_copy(dst_hbm.at[wid], dst_v)
    plsc.subcore_barrier()

    def body(j, carry):
        pltpu.async_copy(hs_hbm.at[src_v.at[j]], rows_v, sem).wait()
        pltpu.sync_copy(rows_v, agg_sh.at[dst_v.at[j]], add=True)
        return carry

    @pl.when(cid == 0)
    def _():
        lax.fori_loop(0, RC0, body, 0)

    @pl.when(cid != 0)
    def _():
        lax.fori_loop(0, RC1, body, 0)

    plsc.subcore_barrier()
    off = cid * NP + sid * ROWS_PER_SUB
    pltpu.sync_copy(agg_sh.at[pl.ds(sid * ROWS_PER_SUB, ROWS_PER_SUB)],
                    out_hbm.at[pl.ds(off, ROWS_PER_SUB)])


@functools.partial(
    pl.kernel,
    out_type=jax.ShapeDtypeStruct((NC * NP, D), jnp.float32),
    mesh=_MESH,
    scratch_types=[
        pltpu.VMEM((STRIDE, CHUNK), jnp.int32),
        pltpu.VMEM((CHUNK, D), jnp.float32),
        pltpu.VMEM_SHARED((NP, D), jnp.float32),
    ],
)
def _sc_deg(dst_hbm, ones_hbm, zeros_hbm, out_hbm, dst_v, ones_v, deg_sh):
    cid = lax.axis_index("c")
    sid = lax.axis_index("s")
    wid = cid * NS + sid
    pltpu.sync_copy(zeros_hbm.at[pl.ds(sid * ROWS_PER_SUB, ROWS_PER_SUB)],
                    deg_sh.at[pl.ds(sid * ROWS_PER_SUB, ROWS_PER_SUB)])
    pltpu.sync_copy(ones_hbm, ones_v)
    pltpu.sync_copy(dst_hbm.at[wid], dst_v)
    plsc.subcore_barrier()

    def body(j, carry):
        pltpu.sync_copy(ones_v, deg_sh.at[dst_v.at[j]], add=True)
        return carry

    @pl.when(cid == 0)
    def _():
        lax.fori_loop(0, RC0, body, 0)

    @pl.when(cid != 0)
    def _():
        lax.fori_loop(0, RC1, body, 0)

    plsc.subcore_barrier()
    off = cid * NP + sid * ROWS_PER_SUB
    pltpu.sync_copy(deg_sh.at[pl.ds(sid * ROWS_PER_SUB, ROWS_PER_SUB)],
                    out_hbm.at[pl.ds(off, ROWS_PER_SUB)])


_R = 512


def _dinv_of(deg2_ref):
    deg = deg2_ref[0, :, 0:1] + deg2_ref[1, :, 0:1] + 1.0
    return lax.rsqrt(deg)


def _pre_body(x_ref, w_ref, deg2_ref, o_ref):
    dinv = _dinv_of(deg2_ref)
    h = jnp.dot(x_ref[...], w_ref[...], preferred_element_type=jnp.float32)
    o_ref[...] = h * dinv


def _mid_body(agg_ref, hs_ref, deg2_ref, w_ref, b_ref, o_ref):
    dinv = _dinv_of(deg2_ref)
    t = (agg_ref[0] + agg_ref[1] + hs_ref[...]) * dinv + b_ref[...]
    t = jnp.maximum(t, 0.0)
    nrm = jnp.sqrt(jnp.sum(t * t, axis=1, keepdims=True))
    t = t / jnp.clip(nrm, 1e-12, None)
    o_ref[...] = jnp.dot(t, w_ref[...], preferred_element_type=jnp.float32) * dinv


def _fin_body(agg_ref, hs_ref, deg2_ref, b_ref, o_ref):
    dinv = _dinv_of(deg2_ref)
    o_ref[...] = (agg_ref[0] + agg_ref[1] + hs_ref[...]) * dinv + b_ref[...]


_ROW_SPEC = pl.BlockSpec((_R, D), lambda i: (i, 0))
_AGG_SPEC = pl.BlockSpec((NC, _R, D), lambda i: (0, i, 0))
_DEG_SPEC = pl.BlockSpec((NC, _R, 8), lambda i: (0, i, 0))
_W_SPEC = pl.BlockSpec((D, D), lambda i: (0, 0))
_B_SPEC = pl.BlockSpec((1, D), lambda i: (0, 0))
_GRID = (NP // _R,)
_OUT = jax.ShapeDtypeStruct((NP, D), jnp.float32)

_tc_pre = pl.pallas_call(
    _pre_body, grid=_GRID, out_shape=_OUT,
    in_specs=[_ROW_SPEC, _W_SPEC, _DEG_SPEC], out_specs=_ROW_SPEC)

_tc_mid = pl.pallas_call(
    _mid_body, grid=_GRID, out_shape=_OUT,
    in_specs=[_AGG_SPEC, _ROW_SPEC, _DEG_SPEC, _W_SPEC, _B_SPEC],
    out_specs=_ROW_SPEC)

_tc_fin = pl.pallas_call(
    _fin_body, grid=_GRID, out_shape=_OUT,
    in_specs=[_AGG_SPEC, _ROW_SPEC, _DEG_SPEC, _B_SPEC], out_specs=_ROW_SPEC)


def _pack_edges(idx):
    e0 = (NW // 2) * RC0 * CHUNK
    e1 = (NW // 2) * RC1 * CHUNK
    c0 = idx[:e0].reshape(NW // 2, RC0, CHUNK)
    c1 = jnp.concatenate(
        [idx[e0:], jnp.full((e0 + e1 - E,), N, jnp.int32)]
    ).reshape(NW // 2, RC1, CHUNK)
    c1 = jnp.concatenate(
        [c1, jnp.full((NW // 2, STRIDE - RC1, CHUNK), N, jnp.int32)], axis=1)
    return jnp.concatenate([c0, c1], axis=0)


def kernel(x, edge_index, W1, b1, W2, b2, W3, b3):
    src = edge_index[0].astype(jnp.int32)
    dst = edge_index[1].astype(jnp.int32)
    src_t = _pack_edges(src)
    dst_t = _pack_edges(dst)
    x_pad = jnp.pad(x, ((0, NP - N), (0, 0)))

    zeros = jnp.zeros((NP, D), jnp.float32)
    onesD = jnp.ones((CHUNK, D), jnp.float32)

    deg2 = _sc_deg(dst_t, onesD, zeros).reshape(NC, NP, D)[:, :, :8]

    hs = _tc_pre(x_pad, W1, deg2)
    agg = _sc_agg(hs, src_t, dst_t, zeros).reshape(NC, NP, D)
    hs = _tc_mid(agg, hs, deg2, W2, b1.reshape(1, D))
    agg = _sc_agg(hs, src_t, dst_t, zeros).reshape(NC, NP, D)
    hs = _tc_mid(agg, hs, deg2, W3, b2.reshape(1, D))
    agg = _sc_agg(hs, src_t, dst_t, zeros).reshape(NC, NP, D)
    out = _tc_fin(agg, hs, deg2, b3.reshape(1, D))
    return out[:N]
